# baseline (speedup 1.0000x reference)
import jax
import jax.numpy as jnp
from jax import lax
from jax.experimental import pallas as pl
from jax.experimental.pallas import tpu as pltpu

_N = 1_000_000
_HIST = 200
_NMAIN = 999_936
_NTAIL = _N - _NMAIN
_HALF = 499_968
_Q = 249_984
_W = 128
_PAD = 1_000_064
_FILL = 16_384
_NFULL = _PAD // _FILL
_FTAIL = _PAD - _NFULL * _FILL
_G = 4


def _body(item_s, rating_s, titem_s, trating_s, out0_h, out1_h,
          buf0, buf1, tl0, tl1, *sems):
    lanes = lax.broadcasted_iota(jnp.int32, (_W,), 0)

    def build(buf, tl, idx_s, val_s, const):
        vec = jnp.full((_FILL,), const, jnp.float32)

        for i in range(_NFULL):
            buf[pl.ds(i * _FILL, _FILL)] = vec
        buf[pl.ds(_NFULL * _FILL, _FTAIL)] = vec[:_FTAIL]

        def one(j):
            ix = idx_s[j]
            b = pl.multiple_of(ix & ~(_W - 1), _W)
            return ix & (_W - 1), val_s[j], b, buf[pl.ds(b, _W)]

        ln0, vl0, b0, w0 = one(0)
        n0 = jnp.where(lanes == ln0, vl0, w0)
        ln1, vl1, b1, w1 = one(1)
        w1 = jnp.where(b1 == b0, n0, w1)
        n1 = jnp.where(lanes == ln1, vl1, w1)
        pend = [(b0, n0), (b1, n1)]
        for j in range(2, _HIST):
            ln, vl, b, w = one(j)
            (qb, qn), (pb, pn) = pend
            buf[pl.ds(qb, _W)] = qn
            w = jnp.where(b == qb, qn, w)
            w = jnp.where(b == pb, pn, w)
            n = jnp.where(lanes == ln, vl, w)
            pend = [(pb, pn), (b, n)]
        for pb, pn in pend:
            buf[pl.ds(pb, _W)] = pn

        tl[...] = buf[pl.ds(_NMAIN, _NTAIL)]

    build(buf0, tl0, item_s, rating_s, 0.0)
    cps0 = [
        pltpu.make_async_copy(buf0.at[pl.ds(q * _Q, _Q)],
                              out0_h.at[pl.ds(q * _Q, _Q)], sems[q])
        for q in range(4)
    ] + [
        pltpu.make_async_copy(tl0, out0_h.at[pl.ds(_NMAIN, _NTAIL)], sems[4]),
    ]
    for cp in cps0:
        cp.start()

    build(buf1, tl1, titem_s, trating_s, float("nan"))
    cps1 = [
        pltpu.make_async_copy(buf1.at[pl.ds(q * _Q, _Q)],
                              out1_h.at[pl.ds(q * _Q, _Q)], sems[5 + q])
        for q in range(4)
    ] + [
        pltpu.make_async_copy(tl1, out1_h.at[pl.ds(_NMAIN, _NTAIL)], sems[9]),
    ]
    for cp in cps1:
        cp.start()

    for cp in cps0 + cps1:
        cp.wait()


_flat_input_tc = pl.pallas_call(
    _body,
    in_specs=[pl.BlockSpec(memory_space=pltpu.SMEM)] * 4,
    out_specs=[pl.BlockSpec(memory_space=pl.ANY)] * 2,
    out_shape=[
        jax.ShapeDtypeStruct((_N,), jnp.float32),
        jax.ShapeDtypeStruct((_N,), jnp.float32),
    ],
    scratch_shapes=[
        pltpu.VMEM((_PAD,), jnp.float32),
        pltpu.VMEM((_PAD,), jnp.float32),
        pltpu.VMEM((_NTAIL,), jnp.float32),
        pltpu.VMEM((_NTAIL,), jnp.float32),
    ] + [pltpu.SemaphoreType.DMA] * 10,
)


@jax.jit
def kernel(item, rating, target_item, target_rating):
    return _flat_input_tc(item.astype(jnp.int32), rating,
                          target_item.astype(jnp.int32), target_rating)


# speedup vs baseline: 1.1150x; 1.1150x over previous
import jax
import jax.numpy as jnp
from jax import lax
from jax.experimental import pallas as pl
from jax.experimental.pallas import tpu as pltpu

_N = 1_000_000
_HIST = 200
_NMAIN = 999_936
_NTAIL = _N - _NMAIN
_HALF = 499_968
_Q = 249_984
_W = 128
_PAD = 1_000_064
_FILL = 16_384
_NFULL = _PAD // _FILL
_FTAIL = _PAD - _NFULL * _FILL
_G = 4


def _body(item_s, rating_s, titem_s, trating_s, out0_h, out1_h,
          buf0, buf1, tl0, tl1, *sems):
    lanes = lax.broadcasted_iota(jnp.int32, (_W,), 0)

    def build(buf, tl, idx_s, val_s, const):
        vec = jnp.full((_FILL,), const, jnp.float32)

        for i in range(_NFULL):
            buf[pl.ds(i * _FILL, _FILL)] = vec
        buf[pl.ds(_NFULL * _FILL, _FTAIL)] = vec[:_FTAIL]

        def one(j):
            ix = idx_s[j]
            b = pl.multiple_of(ix & ~(_W - 1), _W)
            return ix & (_W - 1), val_s[j], b, buf[pl.ds(b, _W)]

        p_lane, p_val, p_base, w = one(0)
        p_new = jnp.where(lanes == p_lane, p_val, w)
        for j in range(1, _HIST):
            ln, vl, b, w = one(j)
            buf[pl.ds(p_base, _W)] = p_new
            w = jnp.where(b == p_base, p_new, w)
            p_base = b
            p_new = jnp.where(lanes == ln, vl, w)
        buf[pl.ds(p_base, _W)] = p_new

        tl[...] = buf[pl.ds(_NMAIN, _NTAIL)]

    build(buf0, tl0, item_s, rating_s, 0.0)
    cps0 = [
        pltpu.make_async_copy(buf0.at[pl.ds(q * _Q, _Q)],
                              out0_h.at[pl.ds(q * _Q, _Q)], sems[q])
        for q in range(4)
    ] + [
        pltpu.make_async_copy(tl0, out0_h.at[pl.ds(_NMAIN, _NTAIL)], sems[4]),
    ]
    for cp in cps0:
        cp.start()

    build(buf1, tl1, titem_s, trating_s, float("nan"))
    cps1 = [
        pltpu.make_async_copy(buf1.at[pl.ds(q * _Q, _Q)],
                              out1_h.at[pl.ds(q * _Q, _Q)], sems[5 + q])
        for q in range(4)
    ] + [
        pltpu.make_async_copy(tl1, out1_h.at[pl.ds(_NMAIN, _NTAIL)], sems[9]),
    ]
    for cp in cps1:
        cp.start()

    for cp in cps0 + cps1:
        cp.wait()


_flat_input_tc = pl.pallas_call(
    _body,
    in_specs=[pl.BlockSpec(memory_space=pltpu.SMEM)] * 4,
    out_specs=[pl.BlockSpec(memory_space=pl.ANY)] * 2,
    out_shape=[
        jax.ShapeDtypeStruct((_N,), jnp.float32),
        jax.ShapeDtypeStruct((_N,), jnp.float32),
    ],
    scratch_shapes=[
        pltpu.VMEM((_PAD,), jnp.float32),
        pltpu.VMEM((_PAD,), jnp.float32),
        pltpu.VMEM((_NTAIL,), jnp.float32),
        pltpu.VMEM((_NTAIL,), jnp.float32),
    ] + [pltpu.SemaphoreType.DMA] * 10,
)


@jax.jit
def kernel(item, rating, target_item, target_rating):
    return _flat_input_tc(item.astype(jnp.int32), rating,
                          target_item.astype(jnp.int32), target_rating)
